# baseline (speedup 1.0000x reference)
import functools

import jax
import jax.numpy as jnp
from jax import lax
from jax.experimental import pallas as pl
from jax.experimental.pallas import tpu as pltpu
from jax.experimental.pallas import tpu_sc as plsc

_B = 16384
_A = 128
_NC = 2
_NS = 16
_NW = _NC * _NS
_RPW = _B // _NW
_G = _RPW // 16


def _sc_rowstats(action_hbm, sum_hbm, max_hbm, buf, sums_v, maxs_v):
    wid = lax.axis_index("s") * _NC + lax.axis_index("c")
    base = wid * _RPW
    pltpu.sync_copy(action_hbm.at[pl.ds(base * _A, _RPW * _A)], buf)
    lanes = jax.lax.iota(jnp.int32, 16)

    def group(g, carry):
        flat0 = (g * 16 + lanes) * _A
        v = plsc.load_gather(buf, [flat0])
        sacc = v
        macc = v
        for j in range(1, _A):
            v = plsc.load_gather(buf, [flat0 + j])
            sacc = sacc + v
            macc = jnp.maximum(macc, v)
        sums_v[pl.ds(g * 16, 16)] = sacc
        maxs_v[pl.ds(g * 16, 16)] = macc
        return carry

    lax.fori_loop(0, _G, group, 0)
    pltpu.sync_copy(sums_v, sum_hbm.at[pl.ds(base, _RPW)])
    pltpu.sync_copy(maxs_v, max_hbm.at[pl.ds(base, _RPW)])


_sc_call = pl.kernel(
    _sc_rowstats,
    out_type=(
        jax.ShapeDtypeStruct((_B,), jnp.float32),
        jax.ShapeDtypeStruct((_B,), jnp.float32),
    ),
    mesh=plsc.VectorSubcoreMesh(core_axis_name="c", subcore_axis_name="s"),
    compiler_params=pltpu.CompilerParams(needs_layout_passes=False),
    scratch_types=[
        pltpu.VMEM((_RPW * _A,), jnp.float32),
        pltpu.VMEM((_RPW,), jnp.float32),
        pltpu.VMEM((_RPW,), jnp.float32),
    ],
)


def _finish_kernel(label_ref, sum_ref, max_ref, out_ref):
    factor = jnp.where(label_ref[0] == 1, jnp.float32(1.25), jnp.float32(0.9))
    s = sum_ref[...] + max_ref[...] * (factor - 1.0)
    ell = jnp.log(s)
    logf = jnp.log(factor)
    a = jnp.float32(_A)
    b = jnp.float32(_B)
    out_ref[0] = (a * jnp.sum(ell * ell) - 2.0 * logf * jnp.sum(ell)
                  + b * logf * logf) / (a * b)


@jax.jit
def _run(action, label_i32):
    rowsum, rowmax = _sc_call(action.reshape(_B * _A))
    out = pl.pallas_call(
        _finish_kernel,
        in_specs=[
            pl.BlockSpec(memory_space=pltpu.SMEM),
            pl.BlockSpec((_B // _A, _A), lambda: (0, 0)),
            pl.BlockSpec((_B // _A, _A), lambda: (0, 0)),
        ],
        out_specs=pl.BlockSpec(memory_space=pltpu.SMEM),
        out_shape=jax.ShapeDtypeStruct((1,), jnp.float32),
    )(label_i32, rowsum.reshape(_B // _A, _A), rowmax.reshape(_B // _A, _A))
    return out[0]


def kernel(action, label):
    return _run(action, label.astype(jnp.int32))


# speedup vs baseline: 1.5036x; 1.5036x over previous
import functools

import jax
import jax.numpy as jnp
from jax import lax
from jax.experimental import pallas as pl
from jax.experimental.pallas import tpu as pltpu
from jax.experimental.pallas import tpu_sc as plsc

_B = 16384
_A = 128
_NC = 2
_NS = 16
_NW = _NC * _NS
_RPW = _B // _NW
_G = _RPW // 16


def _sc_rowstats(action_hbm, sum_hbm, max_hbm, buf, sums_v, maxs_v):
    wid = lax.axis_index("s") * _NC + lax.axis_index("c")
    base = wid * _RPW
    pltpu.sync_copy(action_hbm.at[pl.ds(base * _A, _RPW * _A)], buf)
    lanes = jax.lax.iota(jnp.int32, 16)

    def group(g, carry):
        flat0 = (g * 16 + lanes) * _A + lanes
        v = plsc.load_gather(buf, [flat0])
        sacc = v
        macc = v
        for j in range(1, _A):
            col = (lanes + j) & (_A - 1)
            v = plsc.load_gather(buf, [(g * 16 + lanes) * _A + col])
            sacc = sacc + v
            macc = jnp.maximum(macc, v)
        sums_v[pl.ds(g * 16, 16)] = sacc
        maxs_v[pl.ds(g * 16, 16)] = macc
        return carry

    lax.fori_loop(0, _G, group, 0)
    pltpu.sync_copy(sums_v, sum_hbm.at[pl.ds(base, _RPW)])
    pltpu.sync_copy(maxs_v, max_hbm.at[pl.ds(base, _RPW)])


_sc_call = pl.kernel(
    _sc_rowstats,
    out_type=(
        jax.ShapeDtypeStruct((_B,), jnp.float32),
        jax.ShapeDtypeStruct((_B,), jnp.float32),
    ),
    mesh=plsc.VectorSubcoreMesh(core_axis_name="c", subcore_axis_name="s"),
    compiler_params=pltpu.CompilerParams(needs_layout_passes=False),
    scratch_types=[
        pltpu.VMEM((_RPW * _A,), jnp.float32),
        pltpu.VMEM((_RPW,), jnp.float32),
        pltpu.VMEM((_RPW,), jnp.float32),
    ],
)


def _finish_kernel(label_ref, sum_ref, max_ref, out_ref):
    factor = jnp.where(label_ref[0] == 1, jnp.float32(1.25), jnp.float32(0.9))
    s = sum_ref[...] + max_ref[...] * (factor - 1.0)
    ell = jnp.log(s)
    logf = jnp.log(factor)
    a = jnp.float32(_A)
    b = jnp.float32(_B)
    out_ref[0] = (a * jnp.sum(ell * ell) - 2.0 * logf * jnp.sum(ell)
                  + b * logf * logf) / (a * b)


@jax.jit
def _run(action, label_i32):
    rowsum, rowmax = _sc_call(action.reshape(_B * _A))
    out = pl.pallas_call(
        _finish_kernel,
        in_specs=[
            pl.BlockSpec(memory_space=pltpu.SMEM),
            pl.BlockSpec((_B // _A, _A), lambda: (0, 0)),
            pl.BlockSpec((_B // _A, _A), lambda: (0, 0)),
        ],
        out_specs=pl.BlockSpec(memory_space=pltpu.SMEM),
        out_shape=jax.ShapeDtypeStruct((1,), jnp.float32),
    )(label_i32, rowsum.reshape(_B // _A, _A), rowmax.reshape(_B // _A, _A))
    return out[0]


def kernel(action, label):
    return _run(action, label.astype(jnp.int32))
